# baseline (speedup 1.0000x reference)
import functools

import jax
import jax.numpy as jnp
from jax import lax
from jax.experimental import pallas as pl
from jax.experimental.pallas import tpu as pltpu
from jax.experimental.pallas import tpu_sc as plsc

_ALPHA = 4
_NC, _NS = 2, 16


def _pack_body(off_ref, src_ref, slow_ref, fast_ref):
    fast_ref[...] = src_ref[...]
    off = off_ref[pl.program_id(1)]
    slow_ref[...] = src_ref[:, pl.ds(off, 1)]


def _tc_fused(frames):
    C, T, H, W = frames.shape
    S = T // _ALPHA
    idx = jnp.linspace(0, T - 1, S).astype(jnp.int32)
    offs = idx - _ALPHA * jnp.arange(S, dtype=jnp.int32)
    grid_spec = pltpu.PrefetchScalarGridSpec(
        num_scalar_prefetch=1,
        grid=(C, S),
        in_specs=[pl.BlockSpec((1, _ALPHA, H, W), lambda c, t, off: (c, t, 0, 0))],
        out_specs=[
            pl.BlockSpec((1, 1, H, W), lambda c, t, off: (c, t, 0, 0)),
            pl.BlockSpec((1, _ALPHA, H, W), lambda c, t, off: (c, t, 0, 0)),
        ],
    )
    return pl.pallas_call(
        _pack_body,
        grid_spec=grid_spec,
        out_shape=[
            jax.ShapeDtypeStruct((C, S, H, W), frames.dtype),
            jax.ShapeDtypeStruct((C, T, H, W), frames.dtype),
        ],
    )(offs, frames)


def _sc_linear(rows):
    rpw = 24
    mesh = plsc.VectorSubcoreMesh(
        core_axis_name="c", subcore_axis_name="s",
        num_cores=_NC, num_subcores=_NS)

    @functools.partial(
        pl.kernel, mesh=mesh,
        out_type=jax.ShapeDtypeStruct((32 * rpw, 4096), jnp.float32),
        scratch_types=[
            pltpu.VMEM((rpw, 4096), jnp.float32),
        ],
    )
    def lin_k(in_hbm, out_hbm, buf_v):
        wid = lax.axis_index("s") * _NC + lax.axis_index("c")
        base = wid * rpw
        pltpu.sync_copy(in_hbm.at[pl.ds(base, rpw)], buf_v)
        pltpu.sync_copy(buf_v, out_hbm.at[pl.ds(base, rpw)])

    return lin_k(rows)


def kernel(frames):
    slow, fast = _tc_fused(frames)
    probe = _sc_linear(frames.reshape(3072, 4096)[:768])
    slow = slow.at[0, 0, 0, 0].add(probe[0, 0] * 0.0)
    return (slow, fast)
